# baseline (speedup 1.0000x reference)
import jax
import jax.numpy as jnp
from jax import lax
from jax.experimental import pallas as pl
from jax.experimental.pallas import tpu as pltpu
from jax.experimental.pallas import tpu_sc as plsc

ITEM_COUNT = 1000000
EMBED_DIM = 64
EMBED_NUM = 1024
MAX_LEN = 50
BATCH = 1024

NUM_WORKERS = 32
ROWS = BATCH * MAX_LEN
ROWS_PER_W = ROWS // NUM_WORKERS
CHUNKS = 16
CHUNK = ROWS_PER_W // CHUNKS

BC = 32
RB = BC * MAX_LEN


PAIR_ROWS = ITEM_COUNT // 2

TL = 2048
TGRID = (ITEM_COUNT + TL - 1) // TL


def _tp_body(tt_ref, out_ref):
    x = tt_ref[...]
    t = jnp.swapaxes(x, 0, 1)
    u = t.reshape(TL // 16, 16, EMBED_DIM)
    a = u[:, 0:8, :].reshape(TL // 2, EMBED_DIM)
    b = u[:, 8:16, :].reshape(TL // 2, EMBED_DIM)
    out_ref[...] = jnp.concatenate([a, b], axis=1)


def _tc_transpose_prep(table):
    table_t = table.T
    return pl.pallas_call(
        _tp_body,
        grid=(TGRID,),
        in_specs=[pl.BlockSpec((EMBED_DIM, TL), lambda i: (0, i))],
        out_specs=pl.BlockSpec((TL // 2, 128), lambda i: (i, 0)),
        out_shape=jax.ShapeDtypeStruct((PAIR_ROWS, 128), jnp.float32),
    )(table_t)


HALF = CHUNKS // 2


def _sc_gather_body(table_hbm, ids_hbm, out_hbm, idx_v, rows_v, sem):
    wid = lax.axis_index("s") * 2 + lax.axis_index("c")
    pltpu.sync_copy(ids_hbm.at[wid], idx_v)
    for h in range(2):
        copies = [
            pltpu.async_copy(table_hbm.at[idx_v.at[h * HALF + j]],
                             rows_v.at[j], sem)
            for j in range(HALF)
        ]
        for c in copies:
            c.wait()
        pltpu.sync_copy(rows_v, out_hbm.at[wid, pl.ds(h * HALF, HALF)])


def _sc_gather(table2, ids2):
    mesh = plsc.VectorSubcoreMesh(core_axis_name="c", subcore_axis_name="s")
    fn = pl.kernel(
        _sc_gather_body,
        out_type=jax.ShapeDtypeStruct(
            (NUM_WORKERS, CHUNKS, CHUNK, 128), jnp.float32),
        mesh=mesh,
        scratch_types=[
            pltpu.VMEM((CHUNKS, CHUNK), jnp.int32),
            pltpu.VMEM((HALF, CHUNK, 128), jnp.float32),
            pltpu.SemaphoreType.DMA,
        ],
        compiler_params=pltpu.CompilerParams(use_tc_tiling_on_sc=False),
    )
    return fn(table2, ids2).reshape(ROWS, 128)


def _tc_body(emb2_ref, par_ref, masks_ref, cb_ref, w_ref, b_ref, out_ref):
    emb2 = emb2_ref[...]
    par = par_ref[...]
    emb = jnp.where(par > 0.5, emb2[:, EMBED_DIM:], emb2[:, :EMBED_DIM])
    cb = cb_ref[...]
    x = jnp.dot(emb, w_ref[...], preferred_element_type=jnp.float32) + b_ref[...]
    xc = lax.dot_general(x, cb, (((1,), (1,)), ((), ())),
                         preferred_element_type=jnp.float32)
    x2 = jnp.sum(x * x, axis=1, keepdims=True)
    c2 = jnp.sum(cb * cb, axis=1)
    dist = (x2 + c2[None, :]) - 2.0 * xc
    minval = jnp.min(dist, axis=1, keepdims=True)
    kio = lax.broadcasted_iota(jnp.int32, dist.shape, 1)
    idx = jnp.min(jnp.where(dist == minval, kio, EMBED_NUM),
                  axis=1)
    onehot = (kio == idx[:, None]).astype(jnp.float32)
    rio = lax.broadcasted_iota(jnp.int32, (BC, RB), 0)
    pio = lax.broadcasted_iota(jnp.int32, (BC, RB), 1)
    sel = (pio // MAX_LEN == rio).astype(jnp.float32)
    counts = jnp.dot(sel, onehot, preferred_element_type=jnp.float32)
    q = jnp.dot(counts, cb, preferred_element_type=jnp.float32)
    msum = jnp.sum(masks_ref[...], axis=1, keepdims=True)
    out_ref[...] = q / msum


def _tc_quantize(emb2_flat, parity, masks, code_book, w, b):
    grid = BATCH // BC
    return pl.pallas_call(
        _tc_body,
        grid=(grid,),
        in_specs=[
            pl.BlockSpec((RB, 128), lambda i: (i, 0)),
            pl.BlockSpec((RB, 1), lambda i: (i, 0)),
            pl.BlockSpec((BC, MAX_LEN), lambda i: (i, 0)),
            pl.BlockSpec((EMBED_NUM, EMBED_DIM), lambda i: (0, 0)),
            pl.BlockSpec((EMBED_DIM, EMBED_DIM), lambda i: (0, 0)),
            pl.BlockSpec((1, EMBED_DIM), lambda i: (0, 0)),
        ],
        out_specs=pl.BlockSpec((BC, EMBED_DIM), lambda i: (i, 0)),
        out_shape=jax.ShapeDtypeStruct((BATCH, EMBED_DIM), jnp.float32),
    )(emb2_flat, parity, masks, code_book, w, b)


def kernel(history_item_ids, history_item_masks, embedding_table, code_book,
           W_enc, b_enc):
    table2 = _tc_transpose_prep(embedding_table)
    ids_flat = history_item_ids.reshape(-1)
    ids2 = ((ids_flat // 16) * 8 + ids_flat % 8).reshape(
        NUM_WORKERS, CHUNKS, CHUNK)
    parity = ((ids_flat // 8) % 2).astype(jnp.float32).reshape(ROWS, 1)
    emb2_flat = _sc_gather(table2, ids2)
    return _tc_quantize(emb2_flat, parity, history_item_masks, code_book,
                        W_enc, b_enc.reshape(1, EMBED_DIM))


# speedup vs baseline: 1.2443x; 1.2443x over previous
import jax
import jax.numpy as jnp
from jax import lax
from jax.experimental import pallas as pl
from jax.experimental.pallas import tpu as pltpu
from jax.experimental.pallas import tpu_sc as plsc

ITEM_COUNT = 1000000
EMBED_DIM = 64
EMBED_NUM = 1024
MAX_LEN = 50
BATCH = 1024

NUM_WORKERS = 32
ROWS = BATCH * MAX_LEN
ROWS_PER_W = ROWS // NUM_WORKERS
CHUNKS = 16
CHUNK = ROWS_PER_W // CHUNKS

BC = 64
RB = BC * MAX_LEN


PAIR_ROWS = ITEM_COUNT // 2

TL = 4096
TGRID = (ITEM_COUNT + TL - 1) // TL


def _tp_body(tt_ref, out_ref):
    x = tt_ref[...]
    t = jnp.swapaxes(x, 0, 1)
    u = t.reshape(TL // 16, 16, EMBED_DIM)
    a = u[:, 0:8, :].reshape(TL // 2, EMBED_DIM)
    b = u[:, 8:16, :].reshape(TL // 2, EMBED_DIM)
    out_ref[...] = jnp.concatenate([a, b], axis=1)


def _tc_transpose_prep(table):
    table_t = table.T
    return pl.pallas_call(
        _tp_body,
        grid=(TGRID,),
        in_specs=[pl.BlockSpec((EMBED_DIM, TL), lambda i: (0, i))],
        out_specs=pl.BlockSpec((TL // 2, 128), lambda i: (i, 0)),
        out_shape=jax.ShapeDtypeStruct((PAIR_ROWS, 128), jnp.float32),
    )(table_t)


HALF = CHUNKS // 2


def _sc_gather_body(table_hbm, ids_hbm, out_hbm, idx_v, rows_v, sem):
    wid = lax.axis_index("s") * 2 + lax.axis_index("c")
    pltpu.sync_copy(ids_hbm.at[wid], idx_v)
    for h in range(2):
        copies = [
            pltpu.async_copy(table_hbm.at[idx_v.at[h * HALF + j]],
                             rows_v.at[j], sem)
            for j in range(HALF)
        ]
        for c in copies:
            c.wait()
        pltpu.sync_copy(rows_v, out_hbm.at[wid, pl.ds(h * HALF, HALF)])


def _sc_gather(table2, ids2):
    mesh = plsc.VectorSubcoreMesh(core_axis_name="c", subcore_axis_name="s")
    fn = pl.kernel(
        _sc_gather_body,
        out_type=jax.ShapeDtypeStruct(
            (NUM_WORKERS, CHUNKS, CHUNK, 128), jnp.float32),
        mesh=mesh,
        scratch_types=[
            pltpu.VMEM((CHUNKS, CHUNK), jnp.int32),
            pltpu.VMEM((HALF, CHUNK, 128), jnp.float32),
            pltpu.SemaphoreType.DMA,
        ],
        compiler_params=pltpu.CompilerParams(use_tc_tiling_on_sc=False),
    )
    return fn(table2, ids2).reshape(ROWS, 128)


def _tc_body(emb2_ref, par_ref, masks_ref, cb_ref, w_ref, b_ref, out_ref):
    emb2 = emb2_ref[...]
    par = par_ref[...]
    emb = jnp.where(par > 0.5, emb2[:, EMBED_DIM:], emb2[:, :EMBED_DIM])
    cb = cb_ref[...]
    x = jnp.dot(emb, w_ref[...], preferred_element_type=jnp.float32) + b_ref[...]
    xc = lax.dot_general(x, cb, (((1,), (1,)), ((), ())),
                         preferred_element_type=jnp.float32)
    x2 = jnp.sum(x * x, axis=1, keepdims=True)
    c2 = jnp.sum(cb * cb, axis=1)
    dist = (x2 + c2[None, :]) - 2.0 * xc
    minval = jnp.min(dist, axis=1, keepdims=True)
    kio = lax.broadcasted_iota(jnp.int32, dist.shape, 1)
    idx = jnp.min(jnp.where(dist == minval, kio, EMBED_NUM),
                  axis=1)
    onehot = (kio == idx[:, None]).astype(jnp.float32)
    rio = lax.broadcasted_iota(jnp.int32, (BC, RB), 0)
    pio = lax.broadcasted_iota(jnp.int32, (BC, RB), 1)
    sel = (pio // MAX_LEN == rio).astype(jnp.float32)
    counts = jnp.dot(sel, onehot, preferred_element_type=jnp.float32)
    q = jnp.dot(counts, cb, preferred_element_type=jnp.float32)
    msum = jnp.sum(masks_ref[...], axis=1, keepdims=True)
    out_ref[...] = q / msum


def _tc_quantize(emb2_flat, parity, masks, code_book, w, b):
    grid = BATCH // BC
    return pl.pallas_call(
        _tc_body,
        grid=(grid,),
        in_specs=[
            pl.BlockSpec((RB, 128), lambda i: (i, 0)),
            pl.BlockSpec((RB, 1), lambda i: (i, 0)),
            pl.BlockSpec((BC, MAX_LEN), lambda i: (i, 0)),
            pl.BlockSpec((EMBED_NUM, EMBED_DIM), lambda i: (0, 0)),
            pl.BlockSpec((EMBED_DIM, EMBED_DIM), lambda i: (0, 0)),
            pl.BlockSpec((1, EMBED_DIM), lambda i: (0, 0)),
        ],
        out_specs=pl.BlockSpec((BC, EMBED_DIM), lambda i: (i, 0)),
        out_shape=jax.ShapeDtypeStruct((BATCH, EMBED_DIM), jnp.float32),
    )(emb2_flat, parity, masks, code_book, w, b)


def kernel(history_item_ids, history_item_masks, embedding_table, code_book,
           W_enc, b_enc):
    table2 = _tc_transpose_prep(embedding_table)
    ids_flat = history_item_ids.reshape(-1)
    ids2 = ((ids_flat // 16) * 8 + ids_flat % 8).reshape(
        NUM_WORKERS, CHUNKS, CHUNK)
    parity = ((ids_flat // 8) % 2).astype(jnp.float32).reshape(ROWS, 1)
    emb2_flat = _sc_gather(table2, ids2)
    return _tc_quantize(emb2_flat, parity, history_item_masks, code_book,
                        W_enc, b_enc.reshape(1, EMBED_DIM))


# speedup vs baseline: 1.4245x; 1.1449x over previous
import jax
import jax.numpy as jnp
from jax import lax
from jax.experimental import pallas as pl
from jax.experimental.pallas import tpu as pltpu
from jax.experimental.pallas import tpu_sc as plsc

ITEM_COUNT = 1000000
EMBED_DIM = 64
EMBED_NUM = 1024
MAX_LEN = 50
BATCH = 1024

NUM_WORKERS = 32
ROWS = BATCH * MAX_LEN
ROWS_PER_W = ROWS // NUM_WORKERS
CHUNKS = 16
CHUNK = ROWS_PER_W // CHUNKS

BC = 64
RB = BC * MAX_LEN


PAIR_ROWS = ITEM_COUNT // 2

TL = 8192
TGRID = (ITEM_COUNT + TL - 1) // TL


def _tp_body(tt_ref, out_ref):
    x = tt_ref[...]
    t = jnp.swapaxes(x, 0, 1)
    u = t.reshape(TL // 16, 16, EMBED_DIM)
    a = u[:, 0:8, :].reshape(TL // 2, EMBED_DIM)
    b = u[:, 8:16, :].reshape(TL // 2, EMBED_DIM)
    out_ref[...] = jnp.concatenate([a, b], axis=1)


def _tc_transpose_prep(table):
    table_t = table.T
    return pl.pallas_call(
        _tp_body,
        grid=(TGRID,),
        in_specs=[pl.BlockSpec((EMBED_DIM, TL), lambda i: (0, i))],
        out_specs=pl.BlockSpec((TL // 2, 128), lambda i: (i, 0)),
        out_shape=jax.ShapeDtypeStruct((PAIR_ROWS, 128), jnp.float32),
    )(table_t)


HALF = CHUNKS // 2


def _sc_gather_body(table_hbm, ids_hbm, out_hbm, idx_v, rows_v, sem):
    wid = lax.axis_index("s") * 2 + lax.axis_index("c")
    pltpu.sync_copy(ids_hbm.at[wid], idx_v)
    for h in range(2):
        copies = [
            pltpu.async_copy(table_hbm.at[idx_v.at[h * HALF + j]],
                             rows_v.at[j], sem)
            for j in range(HALF)
        ]
        for c in copies:
            c.wait()
        pltpu.sync_copy(rows_v, out_hbm.at[wid, pl.ds(h * HALF, HALF)])


def _sc_gather(table2, ids2):
    mesh = plsc.VectorSubcoreMesh(core_axis_name="c", subcore_axis_name="s")
    fn = pl.kernel(
        _sc_gather_body,
        out_type=jax.ShapeDtypeStruct(
            (NUM_WORKERS, CHUNKS, CHUNK, 128), jnp.float32),
        mesh=mesh,
        scratch_types=[
            pltpu.VMEM((CHUNKS, CHUNK), jnp.int32),
            pltpu.VMEM((HALF, CHUNK, 128), jnp.float32),
            pltpu.SemaphoreType.DMA,
        ],
        compiler_params=pltpu.CompilerParams(use_tc_tiling_on_sc=False),
    )
    return fn(table2, ids2).reshape(ROWS, 128)


def _tc_body(emb2_ref, par_ref, masks_ref, cb_ref, w_ref, b_ref, out_ref):
    emb2 = emb2_ref[...]
    par = par_ref[...]
    emb = jnp.where(par > 0.5, emb2[:, EMBED_DIM:], emb2[:, :EMBED_DIM])
    cb = cb_ref[...]
    x = jnp.dot(emb, w_ref[...], preferred_element_type=jnp.float32) + b_ref[...]
    xc = lax.dot_general(x, cb, (((1,), (1,)), ((), ())),
                         preferred_element_type=jnp.float32)
    x2 = jnp.sum(x * x, axis=1, keepdims=True)
    c2 = jnp.sum(cb * cb, axis=1)
    dist = (x2 + c2[None, :]) - 2.0 * xc
    minval = jnp.min(dist, axis=1, keepdims=True)
    kio = lax.broadcasted_iota(jnp.int32, dist.shape, 1)
    idx = jnp.min(jnp.where(dist == minval, kio, EMBED_NUM),
                  axis=1)
    onehot = (kio == idx[:, None]).astype(jnp.float32)
    rio = lax.broadcasted_iota(jnp.int32, (BC, RB), 0)
    pio = lax.broadcasted_iota(jnp.int32, (BC, RB), 1)
    sel = (pio // MAX_LEN == rio).astype(jnp.float32)
    counts = jnp.dot(sel, onehot, preferred_element_type=jnp.float32)
    q = jnp.dot(counts, cb, preferred_element_type=jnp.float32)
    msum = jnp.sum(masks_ref[...], axis=1, keepdims=True)
    out_ref[...] = q / msum


def _tc_quantize(emb2_flat, parity, masks, code_book, w, b):
    grid = BATCH // BC
    return pl.pallas_call(
        _tc_body,
        grid=(grid,),
        in_specs=[
            pl.BlockSpec((RB, 128), lambda i: (i, 0)),
            pl.BlockSpec((RB, 1), lambda i: (i, 0)),
            pl.BlockSpec((BC, MAX_LEN), lambda i: (i, 0)),
            pl.BlockSpec((EMBED_NUM, EMBED_DIM), lambda i: (0, 0)),
            pl.BlockSpec((EMBED_DIM, EMBED_DIM), lambda i: (0, 0)),
            pl.BlockSpec((1, EMBED_DIM), lambda i: (0, 0)),
        ],
        out_specs=pl.BlockSpec((BC, EMBED_DIM), lambda i: (i, 0)),
        out_shape=jax.ShapeDtypeStruct((BATCH, EMBED_DIM), jnp.float32),
    )(emb2_flat, parity, masks, code_book, w, b)


def kernel(history_item_ids, history_item_masks, embedding_table, code_book,
           W_enc, b_enc):
    table2 = _tc_transpose_prep(embedding_table)
    ids_flat = history_item_ids.reshape(-1)
    ids2 = ((ids_flat // 16) * 8 + ids_flat % 8).reshape(
        NUM_WORKERS, CHUNKS, CHUNK)
    parity = ((ids_flat // 8) % 2).astype(jnp.float32).reshape(ROWS, 1)
    emb2_flat = _sc_gather(table2, ids2)
    return _tc_quantize(emb2_flat, parity, history_item_masks, code_book,
                        W_enc, b_enc.reshape(1, EMBED_DIM))


# speedup vs baseline: 1.5407x; 1.0816x over previous
import jax
import jax.numpy as jnp
from jax import lax
from jax.experimental import pallas as pl
from jax.experimental.pallas import tpu as pltpu
from jax.experimental.pallas import tpu_sc as plsc

ITEM_COUNT = 1000000
EMBED_DIM = 64
EMBED_NUM = 1024
MAX_LEN = 50
BATCH = 1024

NUM_WORKERS = 32
ROWS = BATCH * MAX_LEN
ROWS_PER_W = ROWS // NUM_WORKERS
CHUNKS = 16
CHUNK = ROWS_PER_W // CHUNKS

BC = 64
RB = BC * MAX_LEN


PAIR_ROWS = ITEM_COUNT // 2

TL = 16384
TGRID = (ITEM_COUNT + TL - 1) // TL


def _tp_body(tt_ref, out_ref):
    x = tt_ref[...]
    t = jnp.swapaxes(x, 0, 1)
    u = t.reshape(TL // 16, 16, EMBED_DIM)
    a = u[:, 0:8, :].reshape(TL // 2, EMBED_DIM)
    b = u[:, 8:16, :].reshape(TL // 2, EMBED_DIM)
    out_ref[...] = jnp.concatenate([a, b], axis=1)


def _tc_transpose_prep(table):
    table_t = table.T
    return pl.pallas_call(
        _tp_body,
        grid=(TGRID,),
        in_specs=[pl.BlockSpec((EMBED_DIM, TL), lambda i: (0, i))],
        out_specs=pl.BlockSpec((TL // 2, 128), lambda i: (i, 0)),
        out_shape=jax.ShapeDtypeStruct((PAIR_ROWS, 128), jnp.float32),
    )(table_t)


HALF = CHUNKS // 2


def _sc_gather_body(table_hbm, ids_hbm, out_hbm, idx_v, rows_v, sem):
    wid = lax.axis_index("s") * 2 + lax.axis_index("c")
    pltpu.sync_copy(ids_hbm.at[wid], idx_v)
    for h in range(2):
        copies = [
            pltpu.async_copy(table_hbm.at[idx_v.at[h * HALF + j]],
                             rows_v.at[j], sem)
            for j in range(HALF)
        ]
        for c in copies:
            c.wait()
        pltpu.sync_copy(rows_v, out_hbm.at[wid, pl.ds(h * HALF, HALF)])


def _sc_gather(table2, ids2):
    mesh = plsc.VectorSubcoreMesh(core_axis_name="c", subcore_axis_name="s")
    fn = pl.kernel(
        _sc_gather_body,
        out_type=jax.ShapeDtypeStruct(
            (NUM_WORKERS, CHUNKS, CHUNK, 128), jnp.float32),
        mesh=mesh,
        scratch_types=[
            pltpu.VMEM((CHUNKS, CHUNK), jnp.int32),
            pltpu.VMEM((HALF, CHUNK, 128), jnp.float32),
            pltpu.SemaphoreType.DMA,
        ],
        compiler_params=pltpu.CompilerParams(use_tc_tiling_on_sc=False),
    )
    return fn(table2, ids2).reshape(ROWS, 128)


def _tc_body(emb2_ref, par_ref, masks_ref, cb_ref, w_ref, b_ref, out_ref):
    emb2 = emb2_ref[...]
    par = par_ref[...]
    emb = jnp.where(par > 0.5, emb2[:, EMBED_DIM:], emb2[:, :EMBED_DIM])
    cb = cb_ref[...]
    x = jnp.dot(emb, w_ref[...], preferred_element_type=jnp.float32) + b_ref[...]
    xc = lax.dot_general(x, cb, (((1,), (1,)), ((), ())),
                         preferred_element_type=jnp.float32)
    x2 = jnp.sum(x * x, axis=1, keepdims=True)
    c2 = jnp.sum(cb * cb, axis=1)
    dist = (x2 + c2[None, :]) - 2.0 * xc
    minval = jnp.min(dist, axis=1, keepdims=True)
    kio = lax.broadcasted_iota(jnp.int32, dist.shape, 1)
    idx = jnp.min(jnp.where(dist == minval, kio, EMBED_NUM),
                  axis=1)
    onehot = (kio == idx[:, None]).astype(jnp.float32)
    rio = lax.broadcasted_iota(jnp.int32, (BC, RB), 0)
    pio = lax.broadcasted_iota(jnp.int32, (BC, RB), 1)
    sel = (pio // MAX_LEN == rio).astype(jnp.float32)
    counts = jnp.dot(sel, onehot, preferred_element_type=jnp.float32)
    q = jnp.dot(counts, cb, preferred_element_type=jnp.float32)
    msum = jnp.sum(masks_ref[...], axis=1, keepdims=True)
    out_ref[...] = q / msum


def _tc_quantize(emb2_flat, parity, masks, code_book, w, b):
    grid = BATCH // BC
    return pl.pallas_call(
        _tc_body,
        grid=(grid,),
        in_specs=[
            pl.BlockSpec((RB, 128), lambda i: (i, 0)),
            pl.BlockSpec((RB, 1), lambda i: (i, 0)),
            pl.BlockSpec((BC, MAX_LEN), lambda i: (i, 0)),
            pl.BlockSpec((EMBED_NUM, EMBED_DIM), lambda i: (0, 0)),
            pl.BlockSpec((EMBED_DIM, EMBED_DIM), lambda i: (0, 0)),
            pl.BlockSpec((1, EMBED_DIM), lambda i: (0, 0)),
        ],
        out_specs=pl.BlockSpec((BC, EMBED_DIM), lambda i: (i, 0)),
        out_shape=jax.ShapeDtypeStruct((BATCH, EMBED_DIM), jnp.float32),
    )(emb2_flat, parity, masks, code_book, w, b)


def kernel(history_item_ids, history_item_masks, embedding_table, code_book,
           W_enc, b_enc):
    table2 = _tc_transpose_prep(embedding_table)
    ids_flat = history_item_ids.reshape(-1)
    ids2 = ((ids_flat // 16) * 8 + ids_flat % 8).reshape(
        NUM_WORKERS, CHUNKS, CHUNK)
    parity = ((ids_flat // 8) % 2).astype(jnp.float32).reshape(ROWS, 1)
    emb2_flat = _sc_gather(table2, ids2)
    return _tc_quantize(emb2_flat, parity, history_item_masks, code_book,
                        W_enc, b_enc.reshape(1, EMBED_DIM))


# speedup vs baseline: 1.6013x; 1.0393x over previous
import jax
import jax.numpy as jnp
from jax import lax
from jax.experimental import pallas as pl
from jax.experimental.pallas import tpu as pltpu
from jax.experimental.pallas import tpu_sc as plsc

ITEM_COUNT = 1000000
EMBED_DIM = 64
EMBED_NUM = 1024
MAX_LEN = 50
BATCH = 1024

NUM_WORKERS = 32
ROWS = BATCH * MAX_LEN
ROWS_PER_W = ROWS // NUM_WORKERS
CHUNKS = 16
CHUNK = ROWS_PER_W // CHUNKS

BC = 64
RB = BC * MAX_LEN


PAIR_ROWS = ITEM_COUNT // 2

TL = 32768
TGRID = (ITEM_COUNT + TL - 1) // TL


def _tp_body(tt_ref, out_ref):
    x = tt_ref[...]
    t = jnp.swapaxes(x, 0, 1)
    u = t.reshape(TL // 16, 16, EMBED_DIM)
    a = u[:, 0:8, :].reshape(TL // 2, EMBED_DIM)
    b = u[:, 8:16, :].reshape(TL // 2, EMBED_DIM)
    out_ref[...] = jnp.concatenate([a, b], axis=1)


def _tc_transpose_prep(table):
    table_t = table.T
    return pl.pallas_call(
        _tp_body,
        grid=(TGRID,),
        in_specs=[pl.BlockSpec((EMBED_DIM, TL), lambda i: (0, i))],
        out_specs=pl.BlockSpec((TL // 2, 128), lambda i: (i, 0)),
        out_shape=jax.ShapeDtypeStruct((PAIR_ROWS, 128), jnp.float32),
    )(table_t)


HALF = CHUNKS // 2


def _sc_gather_body(table_hbm, ids_hbm, out_hbm, idx_v, rows_v, sem):
    wid = lax.axis_index("s") * 2 + lax.axis_index("c")
    pltpu.sync_copy(ids_hbm.at[wid], idx_v)
    for h in range(2):
        copies = [
            pltpu.async_copy(table_hbm.at[idx_v.at[h * HALF + j]],
                             rows_v.at[j], sem)
            for j in range(HALF)
        ]
        for c in copies:
            c.wait()
        pltpu.sync_copy(rows_v, out_hbm.at[wid, pl.ds(h * HALF, HALF)])


def _sc_gather(table2, ids2):
    mesh = plsc.VectorSubcoreMesh(core_axis_name="c", subcore_axis_name="s")
    fn = pl.kernel(
        _sc_gather_body,
        out_type=jax.ShapeDtypeStruct(
            (NUM_WORKERS, CHUNKS, CHUNK, 128), jnp.float32),
        mesh=mesh,
        scratch_types=[
            pltpu.VMEM((CHUNKS, CHUNK), jnp.int32),
            pltpu.VMEM((HALF, CHUNK, 128), jnp.float32),
            pltpu.SemaphoreType.DMA,
        ],
        compiler_params=pltpu.CompilerParams(use_tc_tiling_on_sc=False),
    )
    return fn(table2, ids2).reshape(ROWS, 128)


def _tc_body(emb2_ref, par_ref, masks_ref, cb_ref, w_ref, b_ref, out_ref):
    emb2 = emb2_ref[...]
    par = par_ref[...]
    emb = jnp.where(par > 0.5, emb2[:, EMBED_DIM:], emb2[:, :EMBED_DIM])
    cb = cb_ref[...]
    x = jnp.dot(emb, w_ref[...], preferred_element_type=jnp.float32) + b_ref[...]
    xc = lax.dot_general(x, cb, (((1,), (1,)), ((), ())),
                         preferred_element_type=jnp.float32)
    x2 = jnp.sum(x * x, axis=1, keepdims=True)
    c2 = jnp.sum(cb * cb, axis=1)
    dist = (x2 + c2[None, :]) - 2.0 * xc
    minval = jnp.min(dist, axis=1, keepdims=True)
    kio = lax.broadcasted_iota(jnp.int32, dist.shape, 1)
    idx = jnp.min(jnp.where(dist == minval, kio, EMBED_NUM),
                  axis=1)
    onehot = (kio == idx[:, None]).astype(jnp.float32)
    rio = lax.broadcasted_iota(jnp.int32, (BC, RB), 0)
    pio = lax.broadcasted_iota(jnp.int32, (BC, RB), 1)
    sel = (pio // MAX_LEN == rio).astype(jnp.float32)
    counts = jnp.dot(sel, onehot, preferred_element_type=jnp.float32)
    q = jnp.dot(counts, cb, preferred_element_type=jnp.float32)
    msum = jnp.sum(masks_ref[...], axis=1, keepdims=True)
    out_ref[...] = q / msum


def _tc_quantize(emb2_flat, parity, masks, code_book, w, b):
    grid = BATCH // BC
    return pl.pallas_call(
        _tc_body,
        grid=(grid,),
        in_specs=[
            pl.BlockSpec((RB, 128), lambda i: (i, 0)),
            pl.BlockSpec((RB, 1), lambda i: (i, 0)),
            pl.BlockSpec((BC, MAX_LEN), lambda i: (i, 0)),
            pl.BlockSpec((EMBED_NUM, EMBED_DIM), lambda i: (0, 0)),
            pl.BlockSpec((EMBED_DIM, EMBED_DIM), lambda i: (0, 0)),
            pl.BlockSpec((1, EMBED_DIM), lambda i: (0, 0)),
        ],
        out_specs=pl.BlockSpec((BC, EMBED_DIM), lambda i: (i, 0)),
        out_shape=jax.ShapeDtypeStruct((BATCH, EMBED_DIM), jnp.float32),
    )(emb2_flat, parity, masks, code_book, w, b)


def kernel(history_item_ids, history_item_masks, embedding_table, code_book,
           W_enc, b_enc):
    table2 = _tc_transpose_prep(embedding_table)
    ids_flat = history_item_ids.reshape(-1)
    ids2 = ((ids_flat // 16) * 8 + ids_flat % 8).reshape(
        NUM_WORKERS, CHUNKS, CHUNK)
    parity = ((ids_flat // 8) % 2).astype(jnp.float32).reshape(ROWS, 1)
    emb2_flat = _sc_gather(table2, ids2)
    return _tc_quantize(emb2_flat, parity, history_item_masks, code_book,
                        W_enc, b_enc.reshape(1, EMBED_DIM))
